# baseline (speedup 1.0000x reference)
import functools

import jax
import jax.numpy as jnp
from jax import lax
from jax.experimental import pallas as pl
from jax.experimental.pallas import tpu as pltpu
from jax.experimental.pallas import tpu_sc as plsc

ND = 10000
E = 320000
MSG = 16
H = 48
NR = 2
OUT = 64
NC = 2
NS = 16
NW = NC * NS
CH = 80
NCHUNK = ND // CH
EW = E // NW
ECHUNK = EW // CH

_f32 = jnp.float32
_i32 = jnp.int32

_MESH = plsc.VectorSubcoreMesh(core_axis_name="c", subcore_axis_name="s")
_SC_PARAMS = pltpu.CompilerParams(use_tc_tiling_on_sc=False)


def _w_body(att_ref, basis_ref, cjd_ref, cjs_ref, w_ref, cjdb_ref, cjsb_ref):
    w_ref[...] = jnp.dot(att_ref[...], basis_ref[...],
                         preferred_element_type=_f32)
    cjdb_ref[...] = jnp.broadcast_to(cjd_ref[...], (ND, MSG))
    cjsb_ref[...] = jnp.broadcast_to(cjs_ref[...], (ND, MSG))


def _compute_w(att, basis_flat, cj_drug, cj_dis):
    return pl.pallas_call(
        _w_body,
        out_shape=(jax.ShapeDtypeStruct((NR, ND * MSG), _f32),
                   jax.ShapeDtypeStruct((ND, MSG), _f32),
                   jax.ShapeDtypeStruct((ND, MSG), _f32)),
    )(att, basis_flat, cj_drug, cj_dis)


@functools.partial(
    pl.kernel,
    out_type=jax.ShapeDtypeStruct((4, ND, H), _f32),
    mesh=_MESH,
    compiler_params=_SC_PARAMS,
    scratch_types=[
        pltpu.VMEM((CH,), _i32),
        pltpu.VMEM((CH,), _i32),
        pltpu.VMEM((CH,), _i32),
        pltpu.VMEM((CH, MSG), _f32),
        pltpu.VMEM((CH, MSG), _f32),
        pltpu.VMEM((CH, MSG), _f32),
        pltpu.VMEM((CH, MSG), _f32),
        pltpu.VMEM((CH, H), _f32),
    ],
)
def _sc_build_h(w0, w1, fd0, fd1, fd2, fs0, fs1, fs2, cj_drug, cj_dis,
                h_out, i0, i1, i2, cjv, g0, g1, g2, hbuf):
    cid = lax.axis_index("c")
    sid = lax.axis_index("s")
    wid = cid * NS + sid

    tables = ((fd0, fd1, fd2, cj_drug, w0),
              (fd0, fd1, fd2, cj_drug, w1),
              (fs0, fs1, fs2, cj_dis, w0),
              (fs0, fs1, fs2, cj_dis, w1))

    for t, (f0, f1, f2, cj, w) in enumerate(tables):
        def chunk_body(jj, _, f0=f0, f1=f1, f2=f2, cj=cj, w=w, t=t):
            j = wid + jj * NW

            @pl.when(j < NCHUNK)
            def _():
                off = pl.multiple_of(j * CH, 8)
                pltpu.sync_copy(f0.at[pl.ds(off, CH)], i0)
                pltpu.sync_copy(f1.at[pl.ds(off, CH)], i1)
                pltpu.sync_copy(f2.at[pl.ds(off, CH)], i2)
                pltpu.sync_copy(cj.at[pl.ds(off, CH)], cjv)
                pltpu.sync_copy(w.at[i0], g0)
                pltpu.sync_copy(w.at[i1], g1)
                pltpu.sync_copy(w.at[i2], g2)

                def row(i, _):
                    c = cjv[i, :]
                    hbuf[i, 0:MSG] = g0[i, :] * c
                    hbuf[i, MSG:2 * MSG] = g1[i, :] * c
                    hbuf[i, 2 * MSG:H] = g2[i, :] * c
                    return 0

                lax.fori_loop(0, CH, row, 0)
                pltpu.sync_copy(hbuf, h_out.at[t, pl.ds(off, CH)])
            return 0

        lax.fori_loop(0, (NCHUNK + NW - 1) // NW, chunk_body, 0)


_NBUF = 5


@functools.partial(
    pl.kernel,
    out_type=jax.ShapeDtypeStruct((4, NC, ND, H), _f32),
    mesh=_MESH,
    compiler_params=_SC_PARAMS,
    scratch_types=[
        pltpu.VMEM_SHARED((ND, H), _f32),
        pltpu.VMEM_SHARED((ND, H), _f32),
        pltpu.VMEM((ECHUNK, CH), _i32),
        pltpu.VMEM((ECHUNK, CH), _i32),
        pltpu.VMEM((_NBUF, CH, H), _f32),
        pltpu.VMEM((CH, H), _f32),
        pltpu.SemaphoreType.DMA,
        pltpu.SemaphoreType.DMA,
        pltpu.SemaphoreType.DMA,
        pltpu.SemaphoreType.DMA,
        pltpu.SemaphoreType.DMA,
    ],
)
def _sc_conv(h0, h1, h2, h3, es0, es1, es2, es3, ed0, ed1, ed2, ed3,
             parts, accum, h_sh, srcall, dstall, msg, zbuf, s0, s1, s2, s3, s4):
    cid = lax.axis_index("c")
    sid = lax.axis_index("s")
    gsem = (s0, s1, s2, s3, s4)

    def zrow(i, _):
        z = jnp.zeros((MSG,), _f32)
        zbuf[i, 0:MSG] = z
        zbuf[i, MSG:2 * MSG] = z
        zbuf[i, 2 * MSG:H] = z
        return 0

    lax.fori_loop(0, CH, zrow, 0)

    rowbase = cid * (E // NC // CH) + sid * ECHUNK

    convs = ((h0, es0, ed0), (h1, es1, ed1), (h2, es2, ed2), (h3, es3, ed3))
    for c, (h, es, ed) in enumerate(convs):
        pltpu.sync_copy(es.at[pl.ds(rowbase, ECHUNK)], srcall)
        pltpu.sync_copy(ed.at[pl.ds(rowbase, ECHUNK)], dstall)
        hrow = pl.multiple_of(sid * (ND // NS), 8)
        pltpu.sync_copy(h.at[pl.ds(hrow, ND // NS)],
                        h_sh.at[pl.ds(hrow, ND // NS)])

        def zchunk(jj, _):
            j = sid + jj * NS

            @pl.when(j < NCHUNK)
            def _():
                off = pl.multiple_of(j * CH, 8)
                pltpu.sync_copy(zbuf, accum.at[pl.ds(off, CH)])
            return 0

        lax.fori_loop(0, (NCHUNK + NS - 1) // NS, zchunk, 0)
        plsc.subcore_barrier()

        for b in range(_NBUF):
            pltpu.async_copy(h_sh.at[srcall.at[b]], msg.at[b], gsem[b])

        def egroup(g, _):
            for b in range(_NBUF):
                j = g * _NBUF + b
                pltpu.make_async_copy(h_sh.at[srcall.at[j]], msg.at[b],
                                      gsem[b]).wait()
                pltpu.sync_copy(msg.at[b], accum.at[dstall.at[j]], add=True)

                @pl.when(j + _NBUF < ECHUNK)
                def _(b=b, j=j):
                    pltpu.async_copy(h_sh.at[srcall.at[j + _NBUF]], msg.at[b],
                                     gsem[b])
            return 0

        lax.fori_loop(0, ECHUNK // _NBUF, egroup, 0)
        plsc.subcore_barrier()

        def ochunk(jj, _, c=c):
            j = sid + jj * NS

            @pl.when(j < NCHUNK)
            def _():
                off = pl.multiple_of(j * CH, 8)
                pltpu.sync_copy(accum.at[pl.ds(off, CH)],
                                parts.at[c, cid, pl.ds(off, CH)])
            return 0

        lax.fori_loop(0, (NCHUNK + NS - 1) // NS, ochunk, 0)
        plsc.subcore_barrier()


_RB = 1000


def _post_body(p_ref, ci_ref, w_ref, b_ref, o_ref):
    p = p_ref[...]
    w = w_ref[...]
    b = b_ref[...]
    ci = ci_ref[0]
    hcat = jnp.concatenate([p[0, 0] + p[0, 1], p[1, 0] + p[1, 1]], axis=1)
    o_ref[0] = ci * jnp.dot(hcat, w, preferred_element_type=_f32) + b


def _post(parts, ci2, ufc_w, ufc_b):
    return pl.pallas_call(
        _post_body,
        grid=(2, ND // _RB),
        in_specs=[
            pl.BlockSpec((2, NC, _RB, H), lambda s, r: (s, 0, r, 0)),
            pl.BlockSpec((1, _RB, 1), lambda s, r: (s, r, 0)),
            pl.BlockSpec((NR * H, OUT), lambda s, r: (0, 0)),
            pl.BlockSpec((1, OUT), lambda s, r: (0, 0)),
        ],
        out_specs=pl.BlockSpec((1, _RB, OUT), lambda s, r: (s, r, 0)),
        out_shape=jax.ShapeDtypeStruct((2, ND, OUT), _f32),
    )(parts, ci2, ufc_w, ufc_b.reshape(1, OUT))


def kernel(drug_feat, dis_feat, edge_index_r1, edge_index_r2,
           cj_drug, ci_drug, cj_dis, ci_dis, att, basis, ufc_w, ufc_b):
    w_flat, cjdb, cjsb = _compute_w(att, basis.reshape(4, -1),
                                    cj_drug, cj_dis)
    w3 = w_flat.reshape(NR, ND, MSG)
    w0, w1 = w3[0], w3[1]

    fd0, fd1, fd2 = drug_feat[:, 0], drug_feat[:, 1], drug_feat[:, 2]
    fs0, fs1, fs2 = dis_feat[:, 0], dis_feat[:, 1], dis_feat[:, 2]

    h = _sc_build_h(w0, w1, fd0, fd1, fd2, fs0, fs1, fs2,
                    cjdb, cjsb)

    s1, d1 = (edge_index_r1[0].reshape(E // CH, CH),
              edge_index_r1[1].reshape(E // CH, CH))
    s2, d2 = (edge_index_r2[0].reshape(E // CH, CH),
              edge_index_r2[1].reshape(E // CH, CH))
    parts = _sc_conv(h[0], h[1], h[2], h[3],
                     s1, s2, d1, d2,
                     d1, d2, s1, s2)

    ci2 = jnp.stack([ci_dis, ci_drug], axis=0)
    out = _post(parts, ci2, ufc_w, ufc_b)
    return out[1], out[0]


# speedup vs baseline: 1.0538x; 1.0538x over previous
import functools

import jax
import jax.numpy as jnp
from jax import lax
from jax.experimental import pallas as pl
from jax.experimental.pallas import tpu as pltpu
from jax.experimental.pallas import tpu_sc as plsc

ND = 10000
E = 320000
MSG = 16
H = 48
NR = 2
OUT = 64
NC = 2
NS = 16
NW = NC * NS
CH = 80
NCHUNK = ND // CH
EW = E // NW
ECHUNK = EW // CH

_f32 = jnp.float32
_i32 = jnp.int32

_MESH = plsc.VectorSubcoreMesh(core_axis_name="c", subcore_axis_name="s")
_SC_PARAMS = pltpu.CompilerParams(use_tc_tiling_on_sc=False)


def _w_body(att_ref, basis_ref, cjd_ref, cjs_ref, w_ref, cjdb_ref, cjsb_ref):
    w_ref[...] = jnp.dot(att_ref[...], basis_ref[...],
                         preferred_element_type=_f32)
    cjdb_ref[...] = jnp.broadcast_to(cjd_ref[...], (ND, MSG))
    cjsb_ref[...] = jnp.broadcast_to(cjs_ref[...], (ND, MSG))


def _compute_w(att, basis_flat, cj_drug, cj_dis):
    return pl.pallas_call(
        _w_body,
        out_shape=(jax.ShapeDtypeStruct((NR, ND * MSG), _f32),
                   jax.ShapeDtypeStruct((ND, MSG), _f32),
                   jax.ShapeDtypeStruct((ND, MSG), _f32)),
    )(att, basis_flat, cj_drug, cj_dis)


@functools.partial(
    pl.kernel,
    out_type=jax.ShapeDtypeStruct((4, ND, H), _f32),
    mesh=_MESH,
    compiler_params=_SC_PARAMS,
    scratch_types=[
        pltpu.VMEM((CH,), _i32),
        pltpu.VMEM((CH,), _i32),
        pltpu.VMEM((CH,), _i32),
        pltpu.VMEM((CH, MSG), _f32),
        pltpu.VMEM((CH, MSG), _f32),
        pltpu.VMEM((CH, MSG), _f32),
        pltpu.VMEM((CH, MSG), _f32),
        pltpu.VMEM((CH, H), _f32),
    ],
)
def _sc_build_h(w0, w1, fd0, fd1, fd2, fs0, fs1, fs2, cj_drug, cj_dis,
                h_out, i0, i1, i2, cjv, g0, g1, g2, hbuf):
    cid = lax.axis_index("c")
    sid = lax.axis_index("s")
    wid = cid * NS + sid

    tables = ((fd0, fd1, fd2, cj_drug, w0),
              (fd0, fd1, fd2, cj_drug, w1),
              (fs0, fs1, fs2, cj_dis, w0),
              (fs0, fs1, fs2, cj_dis, w1))

    for t, (f0, f1, f2, cj, w) in enumerate(tables):
        def chunk_body(jj, _, f0=f0, f1=f1, f2=f2, cj=cj, w=w, t=t):
            j = wid + jj * NW

            @pl.when(j < NCHUNK)
            def _():
                off = pl.multiple_of(j * CH, 8)
                pltpu.sync_copy(f0.at[pl.ds(off, CH)], i0)
                pltpu.sync_copy(f1.at[pl.ds(off, CH)], i1)
                pltpu.sync_copy(f2.at[pl.ds(off, CH)], i2)
                pltpu.sync_copy(cj.at[pl.ds(off, CH)], cjv)
                pltpu.sync_copy(w.at[i0], g0)
                pltpu.sync_copy(w.at[i1], g1)
                pltpu.sync_copy(w.at[i2], g2)

                def row(i, _):
                    c = cjv[i, :]
                    hbuf[i, 0:MSG] = g0[i, :] * c
                    hbuf[i, MSG:2 * MSG] = g1[i, :] * c
                    hbuf[i, 2 * MSG:H] = g2[i, :] * c
                    return 0

                lax.fori_loop(0, CH, row, 0)
                pltpu.sync_copy(hbuf, h_out.at[t, pl.ds(off, CH)])
            return 0

        lax.fori_loop(0, (NCHUNK + NW - 1) // NW, chunk_body, 0)


_NBUF = 5


@functools.partial(
    pl.kernel,
    out_type=jax.ShapeDtypeStruct((4, NC, ND, H), _f32),
    mesh=_MESH,
    compiler_params=_SC_PARAMS,
    scratch_types=[
        pltpu.VMEM_SHARED((ND, H), _f32),
        pltpu.VMEM((ECHUNK, CH), _i32),
        pltpu.VMEM((ECHUNK, CH), _i32),
        pltpu.VMEM((_NBUF, CH, H), _f32),
        pltpu.VMEM((CH, H), _f32),
        pltpu.SemaphoreType.DMA,
        pltpu.SemaphoreType.DMA,
        pltpu.SemaphoreType.DMA,
        pltpu.SemaphoreType.DMA,
        pltpu.SemaphoreType.DMA,
        pltpu.SemaphoreType.DMA,
        pltpu.SemaphoreType.DMA,
        pltpu.SemaphoreType.DMA,
        pltpu.SemaphoreType.DMA,
        pltpu.SemaphoreType.DMA,
    ],
)
def _sc_conv(h0, h1, h2, h3, es0, es1, es2, es3, ed0, ed1, ed2, ed3,
             parts, accum, srcall, dstall, msg, zbuf,
             g0s, g1s, g2s, g3s, g4s, s0s, s1s, s2s, s3s, s4s):
    cid = lax.axis_index("c")
    sid = lax.axis_index("s")
    gsem = (g0s, g1s, g2s, g3s, g4s)
    ssem = (s0s, s1s, s2s, s3s, s4s)
    _LAG = 2

    def zrow(i, _):
        z = jnp.zeros((MSG,), _f32)
        zbuf[i, 0:MSG] = z
        zbuf[i, MSG:2 * MSG] = z
        zbuf[i, 2 * MSG:H] = z
        return 0

    lax.fori_loop(0, CH, zrow, 0)

    rowbase = cid * (E // NC // CH) + sid * ECHUNK

    convs = ((h0, es0, ed0), (h1, es1, ed1), (h2, es2, ed2), (h3, es3, ed3))
    for c, (h, es, ed) in enumerate(convs):
        pltpu.sync_copy(es.at[pl.ds(rowbase, ECHUNK)], srcall)
        pltpu.sync_copy(ed.at[pl.ds(rowbase, ECHUNK)], dstall)
        for b in range(_NBUF - 2):
            pltpu.async_copy(h.at[srcall.at[b]], msg.at[b], gsem[b])

        def zchunk(jj, _):
            j = sid + jj * NS

            @pl.when(j < NCHUNK)
            def _():
                off = pl.multiple_of(j * CH, 8)
                pltpu.sync_copy(zbuf, accum.at[pl.ds(off, CH)])
            return 0

        lax.fori_loop(0, (NCHUNK + NS - 1) // NS, zchunk, 0)
        plsc.subcore_barrier()

        for b in range(_NBUF):
            pltpu.make_async_copy(h.at[srcall.at[b]], msg.at[b],
                                  gsem[b]).wait()
            pltpu.async_copy(msg.at[b], accum.at[dstall.at[b]], ssem[b],
                             add=True)
            jn = b + _NBUF - _LAG
            b2 = jn % _NBUF
            if jn >= _NBUF:
                pltpu.make_async_copy(msg.at[b2], accum.at[dstall.at[b2]],
                                      ssem[b2]).wait()
            pltpu.async_copy(h.at[srcall.at[jn]], msg.at[b2], gsem[b2])

        def egroup(g, _, h=h):
            for b in range(_NBUF):
                j = g * _NBUF + b
                pltpu.make_async_copy(h.at[srcall.at[j]], msg.at[b],
                                      gsem[b]).wait()
                pltpu.async_copy(msg.at[b], accum.at[dstall.at[j]], ssem[b],
                                 add=True)

                @pl.when(j + _NBUF - _LAG < ECHUNK)
                def _(b=b, j=j, h=h):
                    b2 = (b + _NBUF - _LAG) % _NBUF
                    pltpu.make_async_copy(msg.at[b2],
                                          accum.at[dstall.at[j - _LAG]],
                                          ssem[b2]).wait()
                    pltpu.async_copy(h.at[srcall.at[j + _NBUF - _LAG]],
                                     msg.at[b2], gsem[b2])
            return 0

        lax.fori_loop(1, ECHUNK // _NBUF, egroup, 0)

        for b in range(_NBUF):
            pltpu.make_async_copy(msg.at[b], accum.at[dstall.at[0]],
                                  ssem[b]).wait()
        plsc.subcore_barrier()

        def ochunk(jj, _, c=c):
            j = sid + jj * NS

            @pl.when(j < NCHUNK)
            def _():
                off = pl.multiple_of(j * CH, 8)
                pltpu.sync_copy(accum.at[pl.ds(off, CH)],
                                parts.at[c, cid, pl.ds(off, CH)])
            return 0

        lax.fori_loop(0, (NCHUNK + NS - 1) // NS, ochunk, 0)
        plsc.subcore_barrier()


_RB = 1000


def _post_body(p_ref, ci_ref, w_ref, b_ref, o_ref):
    p = p_ref[...]
    w = w_ref[...]
    b = b_ref[...]
    ci = ci_ref[0]
    hcat = jnp.concatenate([p[0, 0] + p[0, 1], p[1, 0] + p[1, 1]], axis=1)
    o_ref[0] = ci * jnp.dot(hcat, w, preferred_element_type=_f32) + b


def _post(parts, ci2, ufc_w, ufc_b):
    return pl.pallas_call(
        _post_body,
        grid=(2, ND // _RB),
        in_specs=[
            pl.BlockSpec((2, NC, _RB, H), lambda s, r: (s, 0, r, 0)),
            pl.BlockSpec((1, _RB, 1), lambda s, r: (s, r, 0)),
            pl.BlockSpec((NR * H, OUT), lambda s, r: (0, 0)),
            pl.BlockSpec((1, OUT), lambda s, r: (0, 0)),
        ],
        out_specs=pl.BlockSpec((1, _RB, OUT), lambda s, r: (s, r, 0)),
        out_shape=jax.ShapeDtypeStruct((2, ND, OUT), _f32),
    )(parts, ci2, ufc_w, ufc_b.reshape(1, OUT))


def kernel(drug_feat, dis_feat, edge_index_r1, edge_index_r2,
           cj_drug, ci_drug, cj_dis, ci_dis, att, basis, ufc_w, ufc_b):
    w_flat, cjdb, cjsb = _compute_w(att, basis.reshape(4, -1),
                                    cj_drug, cj_dis)
    w3 = w_flat.reshape(NR, ND, MSG)
    w0, w1 = w3[0], w3[1]

    fd0, fd1, fd2 = drug_feat[:, 0], drug_feat[:, 1], drug_feat[:, 2]
    fs0, fs1, fs2 = dis_feat[:, 0], dis_feat[:, 1], dis_feat[:, 2]

    h = _sc_build_h(w0, w1, fd0, fd1, fd2, fs0, fs1, fs2,
                    cjdb, cjsb)

    s1, d1 = (edge_index_r1[0].reshape(E // CH, CH),
              edge_index_r1[1].reshape(E // CH, CH))
    s2, d2 = (edge_index_r2[0].reshape(E // CH, CH),
              edge_index_r2[1].reshape(E // CH, CH))
    parts = _sc_conv(h[0], h[1], h[2], h[3],
                     s1, s2, d1, d2,
                     d1, d2, s1, s2)

    ci2 = jnp.stack([ci_dis, ci_drug], axis=0)
    out = _post(parts, ci2, ufc_w, ufc_b)
    return out[1], out[0]


# speedup vs baseline: 1.3225x; 1.2550x over previous
import functools

import jax
import jax.numpy as jnp
from jax import lax
from jax.experimental import pallas as pl
from jax.experimental.pallas import tpu as pltpu
from jax.experimental.pallas import tpu_sc as plsc

ND = 10000
E = 320000
MSG = 16
H = 48
NR = 2
OUT = 64
NC = 2
NS = 16
CH = 80
NCHUNK = ND // CH
EW = E // NS
ECHUNK = EW // CH
_NBUF = 5

_f32 = jnp.float32
_i32 = jnp.int32

_MESH = plsc.VectorSubcoreMesh(core_axis_name="c", subcore_axis_name="s")
_SC_PARAMS = pltpu.CompilerParams(use_tc_tiling_on_sc=False)


def _w_body(att_ref, basis_ref, cjd_ref, cjs_ref, w_ref, cjdb_ref, cjsb_ref):
    w_ref[...] = jnp.dot(att_ref[...], basis_ref[...],
                         preferred_element_type=_f32)
    cjdb_ref[...] = jnp.broadcast_to(cjd_ref[...], (ND, MSG))
    cjsb_ref[...] = jnp.broadcast_to(cjs_ref[...], (ND, MSG))


def _compute_w(att, basis_flat, cj_drug, cj_dis):
    return pl.pallas_call(
        _w_body,
        out_shape=(jax.ShapeDtypeStruct((NR, ND * MSG), _f32),
                   jax.ShapeDtypeStruct((ND, MSG), _f32),
                   jax.ShapeDtypeStruct((ND, MSG), _f32)),
    )(att, basis_flat, cj_drug, cj_dis)


@functools.partial(
    pl.kernel,
    out_type=(jax.ShapeDtypeStruct((4, ND, H), _f32),
              jax.ShapeDtypeStruct((4, ND, H), _f32)),
    mesh=_MESH,
    compiler_params=_SC_PARAMS,
    scratch_types=[
        pltpu.VMEM_SHARED((ND, H), _f32),
        pltpu.VMEM((ECHUNK, CH), _i32),
        pltpu.VMEM((ECHUNK, CH), _i32),
        pltpu.VMEM((_NBUF, CH, H), _f32),
        pltpu.VMEM((CH, H), _f32),
        pltpu.VMEM((CH,), _i32),
        pltpu.VMEM((CH,), _i32),
        pltpu.VMEM((CH,), _i32),
        pltpu.VMEM((CH, MSG), _f32),
        pltpu.VMEM((CH, MSG), _f32),
        pltpu.VMEM((CH, MSG), _f32),
        pltpu.VMEM((CH, MSG), _f32),
        pltpu.VMEM((CH, H), _f32),
        pltpu.SemaphoreType.DMA,
        pltpu.SemaphoreType.DMA,
        pltpu.SemaphoreType.DMA,
        pltpu.SemaphoreType.DMA,
        pltpu.SemaphoreType.DMA,
    ],
)
def _sc_main(w0, w1, fd0, fd1, fd2, fs0, fs1, fs2, cjdb, cjsb,
             es0, es1, es2, es3, ed0, ed1, ed2, ed3,
             parts, h_out, accum, srcall, dstall, msg, zbuf,
             i0, i1, i2, cjv, g0, g1, g2, hbuf, s0, s1, s2, s3, s4):
    cid = lax.axis_index("c")
    sid = lax.axis_index("s")
    gsem = (s0, s1, s2, s3, s4)

    def build_table(f0, f1, f2, cj, w, t):
        def chunk_body(jj, _):
            j = sid + jj * NS

            @pl.when(j < NCHUNK)
            def _():
                off = pl.multiple_of(j * CH, 8)
                pltpu.sync_copy(f0.at[pl.ds(off, CH)], i0)
                pltpu.sync_copy(f1.at[pl.ds(off, CH)], i1)
                pltpu.sync_copy(f2.at[pl.ds(off, CH)], i2)
                pltpu.sync_copy(cj.at[pl.ds(off, CH)], cjv)
                pltpu.sync_copy(w.at[i0], g0)
                pltpu.sync_copy(w.at[i1], g1)
                pltpu.sync_copy(w.at[i2], g2)

                def row(i, _):
                    c = cjv[i, :]
                    hbuf[i, 0:MSG] = g0[i, :] * c
                    hbuf[i, MSG:2 * MSG] = g1[i, :] * c
                    hbuf[i, 2 * MSG:H] = g2[i, :] * c
                    return 0

                lax.fori_loop(0, CH, row, 0)
                pltpu.sync_copy(hbuf, h_out.at[t, pl.ds(off, CH)])
            return 0

        lax.fori_loop(0, (NCHUNK + NS - 1) // NS, chunk_body, 0)

    @pl.when(cid == 0)
    def _():
        build_table(fd0, fd1, fd2, cjdb, w0, 0)
        build_table(fd0, fd1, fd2, cjdb, w1, 1)
    @pl.when(cid == 1)
    def _():
        build_table(fs0, fs1, fs2, cjsb, w0, 2)
        build_table(fs0, fs1, fs2, cjsb, w1, 3)

    def zrow(i, _):
        z = jnp.zeros((MSG,), _f32)
        zbuf[i, 0:MSG] = z
        zbuf[i, MSG:2 * MSG] = z
        zbuf[i, 2 * MSG:H] = z
        return 0

    lax.fori_loop(0, CH, zrow, 0)
    plsc.subcore_barrier()

    rowbase = sid * ECHUNK

    def run_conv(t, es, ed):
        pltpu.sync_copy(es.at[pl.ds(rowbase, ECHUNK)], srcall)
        pltpu.sync_copy(ed.at[pl.ds(rowbase, ECHUNK)], dstall)
        hh = h_out.at[t]
        for b in range(_NBUF):
            pltpu.async_copy(hh.at[srcall.at[b]], msg.at[b], gsem[b])

        def zchunk(jj, _):
            j = sid + jj * NS

            @pl.when(j < NCHUNK)
            def _():
                off = pl.multiple_of(j * CH, 8)
                pltpu.sync_copy(zbuf, accum.at[pl.ds(off, CH)])
            return 0

        lax.fori_loop(0, (NCHUNK + NS - 1) // NS, zchunk, 0)
        plsc.subcore_barrier()

        def egroup(g, _):
            for b in range(_NBUF):
                j = g * _NBUF + b
                pltpu.make_async_copy(hh.at[srcall.at[j]], msg.at[b],
                                      gsem[b]).wait()
                pltpu.sync_copy(msg.at[b], accum.at[dstall.at[j]], add=True)

                @pl.when(j + _NBUF < ECHUNK)
                def _(b=b, j=j):
                    pltpu.async_copy(hh.at[srcall.at[j + _NBUF]], msg.at[b],
                                     gsem[b])
            return 0

        lax.fori_loop(0, ECHUNK // _NBUF, egroup, 0)
        plsc.subcore_barrier()

        def ochunk(jj, _):
            j = sid + jj * NS

            @pl.when(j < NCHUNK)
            def _():
                off = pl.multiple_of(j * CH, 8)
                pltpu.sync_copy(accum.at[pl.ds(off, CH)],
                                parts.at[t, pl.ds(off, CH)])
            return 0

        lax.fori_loop(0, (NCHUNK + NS - 1) // NS, ochunk, 0)
        plsc.subcore_barrier()

    @pl.when(cid == 0)
    def _():
        run_conv(0, es0, ed0)
        run_conv(1, es1, ed1)
    @pl.when(cid == 1)
    def _():
        run_conv(2, es2, ed2)
        run_conv(3, es3, ed3)


_RB = 1000


def _post_body(p_ref, ci_ref, w_ref, b_ref, o_ref):
    p = p_ref[...]
    w = w_ref[...]
    b = b_ref[...]
    ci = ci_ref[0]
    hcat = jnp.concatenate([p[0], p[1]], axis=1)
    o_ref[0] = ci * jnp.dot(hcat, w, preferred_element_type=_f32) + b


def _post(parts, ci2, ufc_w, ufc_b):
    return pl.pallas_call(
        _post_body,
        grid=(2, ND // _RB),
        in_specs=[
            pl.BlockSpec((2, _RB, H), lambda s, r: (s, r, 0)),
            pl.BlockSpec((1, _RB, 1), lambda s, r: (s, r, 0)),
            pl.BlockSpec((NR * H, OUT), lambda s, r: (0, 0)),
            pl.BlockSpec((1, OUT), lambda s, r: (0, 0)),
        ],
        out_specs=pl.BlockSpec((1, _RB, OUT), lambda s, r: (s, r, 0)),
        out_shape=jax.ShapeDtypeStruct((2, ND, OUT), _f32),
    )(parts, ci2, ufc_w, ufc_b.reshape(1, OUT))


def kernel(drug_feat, dis_feat, edge_index_r1, edge_index_r2,
           cj_drug, ci_drug, cj_dis, ci_dis, att, basis, ufc_w, ufc_b):
    w_flat, cjdb, cjsb = _compute_w(att, basis.reshape(4, -1),
                                    cj_drug, cj_dis)
    w3 = w_flat.reshape(NR, ND, MSG)
    w0, w1 = w3[0], w3[1]

    fd0, fd1, fd2 = drug_feat[:, 0], drug_feat[:, 1], drug_feat[:, 2]
    fs0, fs1, fs2 = dis_feat[:, 0], dis_feat[:, 1], dis_feat[:, 2]

    s1, d1 = (edge_index_r1[0].reshape(E // CH, CH),
              edge_index_r1[1].reshape(E // CH, CH))
    s2, d2 = (edge_index_r2[0].reshape(E // CH, CH),
              edge_index_r2[1].reshape(E // CH, CH))

    parts, _ = _sc_main(w0, w1, fd0, fd1, fd2, fs0, fs1, fs2, cjdb, cjsb,
                        s1, s2, d1, d2,
                        d1, d2, s1, s2)

    ci2 = jnp.stack([ci_dis, ci_drug], axis=0)
    out = _post(parts, ci2, ufc_w, ufc_b)
    return out[1], out[0]


# speedup vs baseline: 1.4952x; 1.1306x over previous
import functools

import jax
import jax.numpy as jnp
from jax import lax
from jax.experimental import pallas as pl
from jax.experimental.pallas import tpu as pltpu
from jax.experimental.pallas import tpu_sc as plsc

ND = 10000
E = 320000
MSG = 16
H = 48
NR = 2
OUT = 64
NC = 2
NS = 16
CH = 80
NCHUNK = ND // CH
EW = E // NS
ECHUNK = EW // CH
_NBUF = 5

_f32 = jnp.float32
_i32 = jnp.int32

_MESH = plsc.VectorSubcoreMesh(core_axis_name="c", subcore_axis_name="s")
_SC_PARAMS = pltpu.CompilerParams(use_tc_tiling_on_sc=False)


def _w_body(att_ref, basis_ref, cjd_ref, cjs_ref, w_ref, cjdb_ref, cjsb_ref):
    w_ref[...] = jnp.dot(att_ref[...], basis_ref[...],
                         preferred_element_type=_f32)
    cjdb_ref[...] = jnp.broadcast_to(cjd_ref[...], (ND, MSG))
    cjsb_ref[...] = jnp.broadcast_to(cjs_ref[...], (ND, MSG))


def _compute_w(att, basis_flat, cj_drug, cj_dis):
    return pl.pallas_call(
        _w_body,
        out_shape=(jax.ShapeDtypeStruct((NR, ND * MSG), _f32),
                   jax.ShapeDtypeStruct((ND, MSG), _f32),
                   jax.ShapeDtypeStruct((ND, MSG), _f32)),
    )(att, basis_flat, cj_drug, cj_dis)


@functools.partial(
    pl.kernel,
    out_type=(jax.ShapeDtypeStruct((4, ND, H), _f32),
              jax.ShapeDtypeStruct((4, ND, H), _f32)),
    mesh=_MESH,
    compiler_params=_SC_PARAMS,
    scratch_types=[
        pltpu.VMEM_SHARED((ND, H), _f32),
        pltpu.VMEM_SHARED((ND, H), _f32),
        pltpu.VMEM((ECHUNK, CH), _i32),
        pltpu.VMEM((ECHUNK, CH), _i32),
        pltpu.VMEM((_NBUF, CH, H), _f32),
        pltpu.VMEM((CH,), _i32),
        pltpu.VMEM((CH,), _i32),
        pltpu.VMEM((CH,), _i32),
        pltpu.VMEM((CH, MSG), _f32),
        pltpu.VMEM((CH, MSG), _f32),
        pltpu.VMEM((CH, MSG), _f32),
        pltpu.VMEM((CH, MSG), _f32),
        pltpu.SemaphoreType.DMA,
        pltpu.SemaphoreType.DMA,
        pltpu.SemaphoreType.DMA,
        pltpu.SemaphoreType.DMA,
        pltpu.SemaphoreType.DMA,
        pltpu.SemaphoreType.DMA,
    ],
)
def _sc_main(w0, w1, fd0, fd1, fd2, fs0, fs1, fs2, cjdb, cjsb,
             es0, es1, es2, es3, ed0, ed1, ed2, ed3,
             parts, h_out, accA, accB, srcall, dstall, msg,
             i0, i1, i2, cjv, g0, g1, g2, s0, s1, s2, s3, s4, s5):
    cid = lax.axis_index("c")
    sid = lax.axis_index("s")
    gsem = (s0, s1, s2, s3, s4)
    hbuf = msg.at[0]
    zbuf = msg.at[1]

    def build_table(f0, f1, f2, cj, w, t):
        def chunk_body(jj, _):
            j = sid + jj * NS

            @pl.when(j < NCHUNK)
            def _():
                off = pl.multiple_of(j * CH, 8)
                pltpu.async_copy(f0.at[pl.ds(off, CH)], i0, s0)
                pltpu.async_copy(f1.at[pl.ds(off, CH)], i1, s1)
                pltpu.async_copy(f2.at[pl.ds(off, CH)], i2, s2)
                pltpu.async_copy(cj.at[pl.ds(off, CH)], cjv, s3)
                pltpu.make_async_copy(f0.at[pl.ds(off, CH)], i0, s0).wait()
                pltpu.make_async_copy(f1.at[pl.ds(off, CH)], i1, s1).wait()
                pltpu.make_async_copy(f2.at[pl.ds(off, CH)], i2, s2).wait()
                pltpu.async_copy(w.at[i0], g0, s0)
                pltpu.async_copy(w.at[i1], g1, s1)
                pltpu.async_copy(w.at[i2], g2, s2)
                pltpu.make_async_copy(w.at[i0], g0, s0).wait()
                pltpu.make_async_copy(w.at[i1], g1, s1).wait()
                pltpu.make_async_copy(w.at[i2], g2, s2).wait()
                pltpu.make_async_copy(cj.at[pl.ds(off, CH)], cjv, s3).wait()

                def row(i, _):
                    c = cjv[i, :]
                    hbuf[i, 0:MSG] = g0[i, :] * c
                    hbuf[i, MSG:2 * MSG] = g1[i, :] * c
                    hbuf[i, 2 * MSG:H] = g2[i, :] * c
                    return 0

                lax.fori_loop(0, CH, row, 0)
                pltpu.sync_copy(hbuf, h_out.at[t, pl.ds(off, CH)])
            return 0

        lax.fori_loop(0, (NCHUNK + NS - 1) // NS, chunk_body, 0)

    @pl.when(cid == 0)
    def _():
        build_table(fd0, fd1, fd2, cjdb, w0, 0)
        build_table(fd0, fd1, fd2, cjdb, w1, 1)
    @pl.when(cid == 1)
    def _():
        build_table(fs0, fs1, fs2, cjsb, w0, 2)
        build_table(fs0, fs1, fs2, cjsb, w1, 3)

    def zrow(i, _):
        z = jnp.zeros((MSG,), _f32)
        zbuf[i, 0:MSG] = z
        zbuf[i, MSG:2 * MSG] = z
        zbuf[i, 2 * MSG:H] = z
        return 0

    lax.fori_loop(0, CH, zrow, 0)

    def zchunk(jj, _):
        j = sid + jj * NS

        @pl.when(j < NCHUNK)
        def _():
            off = pl.multiple_of(j * CH, 8)
            pltpu.sync_copy(zbuf, accA.at[pl.ds(off, CH)])
            pltpu.sync_copy(zbuf, accB.at[pl.ds(off, CH)])
        return 0

    lax.fori_loop(0, (NCHUNK + NS - 1) // NS, zchunk, 0)
    plsc.subcore_barrier()

    rowbase = sid * ECHUNK
    orow = pl.multiple_of(sid * (ND // NS), 8)

    def run_conv(t, es, ed, accum):
        pltpu.sync_copy(es.at[pl.ds(rowbase, ECHUNK)], srcall)
        pltpu.sync_copy(ed.at[pl.ds(rowbase, ECHUNK)], dstall)
        hh = h_out.at[t]
        for b in range(_NBUF):
            pltpu.async_copy(hh.at[srcall.at[b]], msg.at[b], gsem[b])

        def egroup(g, _):
            for b in range(_NBUF):
                j = g * _NBUF + b
                pltpu.make_async_copy(hh.at[srcall.at[j]], msg.at[b],
                                      gsem[b]).wait()
                pltpu.sync_copy(msg.at[b], accum.at[dstall.at[j]], add=True)

                @pl.when(j + _NBUF < ECHUNK)
                def _(b=b, j=j):
                    pltpu.async_copy(hh.at[srcall.at[j + _NBUF]], msg.at[b],
                                     gsem[b])
            return 0

        lax.fori_loop(0, ECHUNK // _NBUF, egroup, 0)
        plsc.subcore_barrier()

    def core_work(t0, e0, e1):
        run_conv(t0, e0[0], e0[1], accA)
        pltpu.async_copy(accA.at[pl.ds(orow, ND // NS)],
                         parts.at[t0, pl.ds(orow, ND // NS)], s5)
        run_conv(t0 + 1, e1[0], e1[1], accB)
        pltpu.make_async_copy(accA.at[pl.ds(orow, ND // NS)],
                              parts.at[t0, pl.ds(orow, ND // NS)], s5).wait()
        pltpu.sync_copy(accB.at[pl.ds(orow, ND // NS)],
                        parts.at[t0 + 1, pl.ds(orow, ND // NS)])

    @pl.when(cid == 0)
    def _():
        core_work(0, (es0, ed0), (es1, ed1))
    @pl.when(cid == 1)
    def _():
        core_work(2, (es2, ed2), (es3, ed3))


_RB = 1000


def _post_body(p_ref, ci_ref, w_ref, b_ref, o_ref):
    p = p_ref[...]
    w = w_ref[...]
    b = b_ref[...]
    ci = ci_ref[0]
    hcat = jnp.concatenate([p[0], p[1]], axis=1)
    o_ref[0] = ci * jnp.dot(hcat, w, preferred_element_type=_f32) + b


def _post(parts, ci2, ufc_w, ufc_b):
    return pl.pallas_call(
        _post_body,
        grid=(2, ND // _RB),
        in_specs=[
            pl.BlockSpec((2, _RB, H), lambda s, r: (s, r, 0)),
            pl.BlockSpec((1, _RB, 1), lambda s, r: (s, r, 0)),
            pl.BlockSpec((NR * H, OUT), lambda s, r: (0, 0)),
            pl.BlockSpec((1, OUT), lambda s, r: (0, 0)),
        ],
        out_specs=pl.BlockSpec((1, _RB, OUT), lambda s, r: (s, r, 0)),
        out_shape=jax.ShapeDtypeStruct((2, ND, OUT), _f32),
    )(parts, ci2, ufc_w, ufc_b.reshape(1, OUT))


def kernel(drug_feat, dis_feat, edge_index_r1, edge_index_r2,
           cj_drug, ci_drug, cj_dis, ci_dis, att, basis, ufc_w, ufc_b):
    w_flat, cjdb, cjsb = _compute_w(att, basis.reshape(4, -1),
                                    cj_drug, cj_dis)
    w3 = w_flat.reshape(NR, ND, MSG)
    w0, w1 = w3[0], w3[1]

    fd0, fd1, fd2 = drug_feat[:, 0], drug_feat[:, 1], drug_feat[:, 2]
    fs0, fs1, fs2 = dis_feat[:, 0], dis_feat[:, 1], dis_feat[:, 2]

    s1, d1 = (edge_index_r1[0].reshape(E // CH, CH),
              edge_index_r1[1].reshape(E // CH, CH))
    s2, d2 = (edge_index_r2[0].reshape(E // CH, CH),
              edge_index_r2[1].reshape(E // CH, CH))

    parts, _ = _sc_main(w0, w1, fd0, fd1, fd2, fs0, fs1, fs2, cjdb, cjsb,
                        s1, s2, d1, d2,
                        d1, d2, s1, s2)

    ci2 = jnp.stack([ci_dis, ci_drug], axis=0)
    out = _post(parts, ci2, ufc_w, ufc_b)
    return out[1], out[0]
